# baseline (speedup 1.0000x reference)
import functools

import jax
import jax.numpy as jnp
from jax import lax
from jax.experimental import pallas as pl
from jax.experimental.pallas import tpu as pltpu
from jax.experimental.pallas import tpu_sc as plsc

_NUM_DAYS = 366
_B = 128
_P = (121, 240)
_NW = 32
_SC_ROWS = 96
_TC_ROWS = _B - _SC_ROWS
_ROWS_PER_W = _SC_ROWS // _NW
_REG = 1e-09
_CHUNKS = ((0, 64), (64, 57))
_DEPTH = len(_CHUNKS)


def _sc_body(x0, x1, x2, x3, md_hbm,
             w0, b0, w1, b1, w2, b2, w3, b3,
             o0, o1, o2, o3,
             md_v,
             wp0, bp0, xp0, wp1, bp1, xp1,
             sw0, sb0, sx0, so0, sw1, sb1, sx1, so1):
    c = lax.axis_index("c")
    s = lax.axis_index("s")
    wid = s * 2 + c
    pltpu.sync_copy(md_hbm, md_v)

    keys = []
    for k in range(_ROWS_PER_W):
        b = wid * _ROWS_PER_W + k
        vals = md_v[pl.ds(pl.multiple_of(b * 8, 8), 16)]
        keys.append((b, vals[0]))

    stages = []
    for k in range(_ROWS_PER_W):
        for ch in range(2):
            for grp in ((x0, w0, b0, o0, False),
                        (x1, w1, b1, o1, True),
                        (x2, w2, b2, o2, False),
                        (x3, w3, b3, o3, True)):
                for chunk in _CHUNKS:
                    stages.append((k, ch, grp, chunk))

    bufs = ((wp0, bp0, xp0, sw0, sb0, sx0, so0),
            (wp1, bp1, xp1, sw1, sb1, sx1, so1))
    in_copies = [None] * _DEPTH
    out_copies = [None] * _DEPTH

    def issue(t):
        k, ch, (x_hbm, w_hbm, b_hbm, _, _), (r0, nr) = stages[t]
        b, d = keys[k]
        wpl, bpl, xpl, sw, sb, sx, _ = bufs[t % _DEPTH]
        if out_copies[t % _DEPTH] is not None:
            out_copies[t % _DEPTH].wait()
            out_copies[t % _DEPTH] = None
        cw = pltpu.async_copy(w_hbm.at[pl.ds(d, 1), ch, pl.ds(r0, nr)],
                              wpl, sw)
        cb = pltpu.async_copy(b_hbm.at[pl.ds(d, 1), ch, pl.ds(r0, nr)],
                              bpl, sb)
        cx = pltpu.async_copy(x_hbm.at[pl.ds(b, 1), ch, pl.ds(r0, nr)],
                              xpl, sx)
        in_copies[t % _DEPTH] = (cw, cb, cx)

    n_stages = len(stages)
    for t in range(_DEPTH - 1):
        issue(t)
    for t in range(n_stages):
        if t + _DEPTH - 1 < n_stages:
            issue(t + _DEPTH - 1)
        for cpy in in_copies[t % _DEPTH]:
            cpy.wait()
        k, ch, (_, _, _, out_hbm, clip), (r0, nr) = stages[t]
        b, _ = keys[k]
        wpl, bpl, xpl, _, _, _, so = bufs[t % _DEPTH]

        def row_body(r, carry, clip=clip, wpl=wpl, bpl=bpl, xpl=xpl):
            for i in range(_P[1] // 16):
                o = i * 16
                v = (bpl[0, r, pl.ds(o, 16)]
                     + wpl[0, r, pl.ds(o, 16)] * xpl[0, r, pl.ds(o, 16)])
                if clip:
                    v = jnp.maximum(v, _REG)
                xpl[0, r, pl.ds(o, 16)] = v
            return carry

        lax.fori_loop(0, nr, row_body, 0)
        out_copies[t % _DEPTH] = pltpu.async_copy(
            xpl, out_hbm.at[pl.ds(b, 1), ch, pl.ds(r0, nr)], so)

    for oc in out_copies:
        oc.wait()


_sc_kernel = functools.partial(
    pl.kernel,
    out_type=[jax.ShapeDtypeStruct((_SC_ROWS, 2) + _P, jnp.float32)] * 4,
    mesh=plsc.VectorSubcoreMesh(core_axis_name="c", subcore_axis_name="s"),
    compiler_params=pltpu.CompilerParams(use_tc_tiling_on_sc=True),
    scratch_types=[
        pltpu.VMEM((_B * 8 + 16,), jnp.int32),
        pltpu.VMEM((1, _CHUNKS[0][1], _P[1]), jnp.float32),
        pltpu.VMEM((1, _CHUNKS[0][1], _P[1]), jnp.float32),
        pltpu.VMEM((1, _CHUNKS[0][1], _P[1]), jnp.float32),
        pltpu.VMEM((1, _CHUNKS[1][1], _P[1]), jnp.float32),
        pltpu.VMEM((1, _CHUNKS[1][1], _P[1]), jnp.float32),
        pltpu.VMEM((1, _CHUNKS[1][1], _P[1]), jnp.float32),
    ] + [pltpu.SemaphoreType.DMA] * 8,
)(_sc_body)


def _tc_body(md_ref, x0, x1, x2, x3,
             w0, b0, w1, b1, w2, b2, w3, b3,
             o0, o1, o2, o3):
    del md_ref
    o0[...] = b0[...] + w0[...] * x0[...]
    o1[...] = jnp.maximum(b1[...] + w1[...] * x1[...], _REG)
    o2[...] = b2[...] + w2[...] * x2[...]
    o3[...] = jnp.maximum(b3[...] + w3[...] * x3[...], _REG)


def _x_spec():
    return pl.BlockSpec((1, 2) + _P, lambda i, md: (i + _SC_ROWS, 0, 0, 0))


def _tab_spec():
    return pl.BlockSpec((1, 2) + _P, lambda i, md: (md[i + _SC_ROWS], 0, 0, 0))


_tc_kernel = pl.pallas_call(
    _tc_body,
    grid_spec=pltpu.PrefetchScalarGridSpec(
        num_scalar_prefetch=1,
        grid=(_TC_ROWS,),
        in_specs=[_x_spec(), _x_spec(), _x_spec(), _x_spec(),
                  _tab_spec(), _tab_spec(), _tab_spec(), _tab_spec(),
                  _tab_spec(), _tab_spec(), _tab_spec(), _tab_spec()],
        out_specs=[pl.BlockSpec((1, 2) + _P, lambda i, md: (i, 0, 0, 0))] * 4,
    ),
    out_shape=[jax.ShapeDtypeStruct((_TC_ROWS, 2) + _P, jnp.float32)] * 4,
)


def kernel(model_parameters_t2m_mu, model_parameters_t2m_sigma,
           model_parameters_tp_cube_root_mu, model_parameters_tp_cube_root_sigma,
           monthday,
           w_t2m_loc, b_t2m_loc, w_t2m_scale, b_t2m_scale,
           w_tp_loc, b_tp_loc, w_tp_scale, b_tp_scale):
    md32 = monthday.astype(jnp.int32)
    md = jnp.concatenate([jnp.repeat(md32, 8), jnp.zeros((16,), jnp.int32)])

    sc_outs = _sc_kernel(
        model_parameters_t2m_mu, model_parameters_t2m_sigma,
        model_parameters_tp_cube_root_mu, model_parameters_tp_cube_root_sigma,
        md,
        w_t2m_loc, b_t2m_loc, w_t2m_scale, b_t2m_scale,
        w_tp_loc, b_tp_loc, w_tp_scale, b_tp_scale)

    tc_outs = _tc_kernel(
        md32,
        model_parameters_t2m_mu, model_parameters_t2m_sigma,
        model_parameters_tp_cube_root_mu, model_parameters_tp_cube_root_sigma,
        w_t2m_loc, b_t2m_loc, w_t2m_scale, b_t2m_scale,
        w_tp_loc, b_tp_loc, w_tp_scale, b_tp_scale)

    return tuple(jnp.concatenate([so, to], axis=0)
                 for so, to in zip(sc_outs, tc_outs))


# speedup vs baseline: 1.0831x; 1.0831x over previous
import functools

import jax
import jax.numpy as jnp
from jax import lax
from jax.experimental import pallas as pl
from jax.experimental.pallas import tpu as pltpu
from jax.experimental.pallas import tpu_sc as plsc

_NUM_DAYS = 366
_B = 128
_P = (121, 240)
_NW = 32
_ROWS_PER_W = _B // _NW
_REG = 1e-09
_CHUNKS = ((0, 64), (64, 57))
_DEPTH = len(_CHUNKS)


def _sc_body(x1, x3, md_hbm,
             w1, b1, w3, b3,
             o1, o3,
             md_v,
             wp0, bp0, xp0, wp1, bp1, xp1,
             sw0, sb0, sx0, so0, sw1, sb1, sx1, so1):
    c = lax.axis_index("c")
    s = lax.axis_index("s")
    wid = s * 2 + c
    pltpu.sync_copy(md_hbm, md_v)

    keys = []
    for k in range(_ROWS_PER_W):
        b = wid * _ROWS_PER_W + k
        vals = md_v[pl.ds(pl.multiple_of(b * 8, 8), 16)]
        keys.append((b, vals[0]))

    stages = []
    for k in range(_ROWS_PER_W):
        for ch in range(2):
            for grp in ((x1, w1, b1, o1),
                        (x3, w3, b3, o3)):
                for chunk in _CHUNKS:
                    stages.append((k, ch, grp, chunk))

    bufs = ((wp0, bp0, xp0, sw0, sb0, sx0, so0),
            (wp1, bp1, xp1, sw1, sb1, sx1, so1))
    in_copies = [None] * _DEPTH
    out_copies = [None] * _DEPTH

    def issue(t):
        k, ch, (x_hbm, w_hbm, b_hbm, _), (r0, nr) = stages[t]
        b, d = keys[k]
        wpl, bpl, xpl, sw, sb, sx, _ = bufs[t % _DEPTH]
        if out_copies[t % _DEPTH] is not None:
            out_copies[t % _DEPTH].wait()
            out_copies[t % _DEPTH] = None
        cw = pltpu.async_copy(w_hbm.at[pl.ds(d, 1), ch, pl.ds(r0, nr)],
                              wpl, sw)
        cb = pltpu.async_copy(b_hbm.at[pl.ds(d, 1), ch, pl.ds(r0, nr)],
                              bpl, sb)
        cx = pltpu.async_copy(x_hbm.at[pl.ds(b, 1), ch, pl.ds(r0, nr)],
                              xpl, sx)
        in_copies[t % _DEPTH] = (cw, cb, cx)

    n_stages = len(stages)
    for t in range(_DEPTH - 1):
        issue(t)
    for t in range(n_stages):
        if t + _DEPTH - 1 < n_stages:
            issue(t + _DEPTH - 1)
        for cpy in in_copies[t % _DEPTH]:
            cpy.wait()
        k, ch, (_, _, _, out_hbm), (r0, nr) = stages[t]
        b, _ = keys[k]
        wpl, bpl, xpl, _, _, _, so = bufs[t % _DEPTH]

        def row_body(r, carry, wpl=wpl, bpl=bpl, xpl=xpl):
            for i in range(_P[1] // 16):
                o = i * 16
                v = (bpl[0, r, pl.ds(o, 16)]
                     + wpl[0, r, pl.ds(o, 16)] * xpl[0, r, pl.ds(o, 16)])
                v = jnp.maximum(v, _REG)
                xpl[0, r, pl.ds(o, 16)] = v
            return carry

        lax.fori_loop(0, nr, row_body, 0)
        out_copies[t % _DEPTH] = pltpu.async_copy(
            xpl, out_hbm.at[pl.ds(b, 1), ch, pl.ds(r0, nr)], so)

    for oc in out_copies:
        oc.wait()


_sc_kernel = functools.partial(
    pl.kernel,
    out_type=[jax.ShapeDtypeStruct((_B, 2) + _P, jnp.float32)] * 2,
    mesh=plsc.VectorSubcoreMesh(core_axis_name="c", subcore_axis_name="s"),
    compiler_params=pltpu.CompilerParams(use_tc_tiling_on_sc=True),
    scratch_types=[
        pltpu.VMEM((_B * 8 + 16,), jnp.int32),
        pltpu.VMEM((1, _CHUNKS[0][1], _P[1]), jnp.float32),
        pltpu.VMEM((1, _CHUNKS[0][1], _P[1]), jnp.float32),
        pltpu.VMEM((1, _CHUNKS[0][1], _P[1]), jnp.float32),
        pltpu.VMEM((1, _CHUNKS[1][1], _P[1]), jnp.float32),
        pltpu.VMEM((1, _CHUNKS[1][1], _P[1]), jnp.float32),
        pltpu.VMEM((1, _CHUNKS[1][1], _P[1]), jnp.float32),
    ] + [pltpu.SemaphoreType.DMA] * 8,
)(_sc_body)


def _tc_body(md_ref, x0, x2, w0, b0, w2, b2, o0, o2):
    del md_ref
    o0[...] = b0[...] + w0[...] * x0[...]
    o2[...] = b2[...] + w2[...] * x2[...]


def _x_spec():
    return pl.BlockSpec((1, 2) + _P, lambda i, md: (i, 0, 0, 0))


def _tab_spec():
    return pl.BlockSpec((1, 2) + _P, lambda i, md: (md[i], 0, 0, 0))


_tc_kernel = pl.pallas_call(
    _tc_body,
    grid_spec=pltpu.PrefetchScalarGridSpec(
        num_scalar_prefetch=1,
        grid=(_B,),
        in_specs=[_x_spec(), _x_spec(),
                  _tab_spec(), _tab_spec(), _tab_spec(), _tab_spec()],
        out_specs=[pl.BlockSpec((1, 2) + _P, lambda i, md: (i, 0, 0, 0))] * 2,
    ),
    out_shape=[jax.ShapeDtypeStruct((_B, 2) + _P, jnp.float32)] * 2,
)


def kernel(model_parameters_t2m_mu, model_parameters_t2m_sigma,
           model_parameters_tp_cube_root_mu, model_parameters_tp_cube_root_sigma,
           monthday,
           w_t2m_loc, b_t2m_loc, w_t2m_scale, b_t2m_scale,
           w_tp_loc, b_tp_loc, w_tp_scale, b_tp_scale):
    md32 = monthday.astype(jnp.int32)
    md = jnp.concatenate([jnp.repeat(md32, 8), jnp.zeros((16,), jnp.int32)])

    t2m_scale, tp_scale = _sc_kernel(
        model_parameters_t2m_sigma, model_parameters_tp_cube_root_sigma,
        md,
        w_t2m_scale, b_t2m_scale, w_tp_scale, b_tp_scale)

    t2m_loc, tp_loc = _tc_kernel(
        md32,
        model_parameters_t2m_mu, model_parameters_tp_cube_root_mu,
        w_t2m_loc, b_t2m_loc, w_tp_loc, b_tp_loc)

    return (t2m_loc, t2m_scale, tp_loc, tp_scale)


# speedup vs baseline: 1.1199x; 1.0339x over previous
import functools

import jax
import jax.numpy as jnp
from jax import lax
from jax.experimental import pallas as pl
from jax.experimental.pallas import tpu as pltpu
from jax.experimental.pallas import tpu_sc as plsc

_NUM_DAYS = 366
_B = 128
_P = (121, 240)
_NW = 32
_ROWS_PER_W = _B // _NW
_REG = 1e-09
_CHUNKS = ((0, 64), (64, 57))
_DEPTH = len(_CHUNKS)


def _sc_body(clip, xa, xb, md_hbm,
             wa, ba, wb, bb,
             oa, ob,
             md_v,
             wp0, bp0, xp0, wp1, bp1, xp1,
             sw0, sb0, sx0, so0, sw1, sb1, sx1, so1):
    c = lax.axis_index("c")
    s = lax.axis_index("s")
    wid = s * 2 + c
    pltpu.sync_copy(md_hbm, md_v)

    keys = []
    for k in range(_ROWS_PER_W):
        b = wid * _ROWS_PER_W + k
        vals = md_v[pl.ds(pl.multiple_of(b * 8, 8), 16)]
        keys.append((b, vals[0]))

    stages = []
    for k in range(_ROWS_PER_W):
        for ch in range(2):
            for grp in ((xa, wa, ba, oa),
                        (xb, wb, bb, ob)):
                for chunk in _CHUNKS:
                    stages.append((k, ch, grp, chunk))

    bufs = ((wp0, bp0, xp0, sw0, sb0, sx0, so0),
            (wp1, bp1, xp1, sw1, sb1, sx1, so1))
    in_copies = [None] * _DEPTH
    out_copies = [None] * _DEPTH

    def issue(t):
        k, ch, (x_hbm, w_hbm, b_hbm, _), (r0, nr) = stages[t]
        b, d = keys[k]
        wpl, bpl, xpl, sw, sb, sx, _ = bufs[t % _DEPTH]
        if out_copies[t % _DEPTH] is not None:
            out_copies[t % _DEPTH].wait()
            out_copies[t % _DEPTH] = None
        cw = pltpu.async_copy(w_hbm.at[pl.ds(d, 1), ch, pl.ds(r0, nr)],
                              wpl, sw)
        cb = pltpu.async_copy(b_hbm.at[pl.ds(d, 1), ch, pl.ds(r0, nr)],
                              bpl, sb)
        cx = pltpu.async_copy(x_hbm.at[pl.ds(b, 1), ch, pl.ds(r0, nr)],
                              xpl, sx)
        in_copies[t % _DEPTH] = (cw, cb, cx)

    n_stages = len(stages)
    for t in range(_DEPTH - 1):
        issue(t)
    for t in range(n_stages):
        if t + _DEPTH - 1 < n_stages:
            issue(t + _DEPTH - 1)
        for cpy in in_copies[t % _DEPTH]:
            cpy.wait()
        k, ch, (_, _, _, out_hbm), (r0, nr) = stages[t]
        b, _ = keys[k]
        wpl, bpl, xpl, _, _, _, so = bufs[t % _DEPTH]

        def row_body(r, carry, wpl=wpl, bpl=bpl, xpl=xpl):
            for i in range(_P[1] // 16):
                o = i * 16
                v = (bpl[0, r, pl.ds(o, 16)]
                     + wpl[0, r, pl.ds(o, 16)] * xpl[0, r, pl.ds(o, 16)])
                if clip:
                    v = jnp.maximum(v, _REG)
                xpl[0, r, pl.ds(o, 16)] = v
            return carry

        lax.fori_loop(0, nr, row_body, 0)
        out_copies[t % _DEPTH] = pltpu.async_copy(
            xpl, out_hbm.at[pl.ds(b, 1), ch, pl.ds(r0, nr)], so)

    for oc in out_copies:
        oc.wait()


def _make_sc_kernel(clip):
    return functools.partial(
        pl.kernel,
        out_type=[jax.ShapeDtypeStruct((_B, 2) + _P, jnp.float32)] * 2,
        mesh=plsc.VectorSubcoreMesh(core_axis_name="c",
                                    subcore_axis_name="s"),
        compiler_params=pltpu.CompilerParams(use_tc_tiling_on_sc=True),
        scratch_types=[
            pltpu.VMEM((_B * 8 + 16,), jnp.int32),
            pltpu.VMEM((1, _CHUNKS[0][1], _P[1]), jnp.float32),
            pltpu.VMEM((1, _CHUNKS[0][1], _P[1]), jnp.float32),
            pltpu.VMEM((1, _CHUNKS[0][1], _P[1]), jnp.float32),
            pltpu.VMEM((1, _CHUNKS[1][1], _P[1]), jnp.float32),
            pltpu.VMEM((1, _CHUNKS[1][1], _P[1]), jnp.float32),
            pltpu.VMEM((1, _CHUNKS[1][1], _P[1]), jnp.float32),
        ] + [pltpu.SemaphoreType.DMA] * 8,
    )(functools.partial(_sc_body, clip))


_sc_loc_kernel = _make_sc_kernel(False)
_sc_scale_kernel = _make_sc_kernel(True)


def kernel(model_parameters_t2m_mu, model_parameters_t2m_sigma,
           model_parameters_tp_cube_root_mu, model_parameters_tp_cube_root_sigma,
           monthday,
           w_t2m_loc, b_t2m_loc, w_t2m_scale, b_t2m_scale,
           w_tp_loc, b_tp_loc, w_tp_scale, b_tp_scale):
    md = jnp.concatenate([jnp.repeat(monthday.astype(jnp.int32), 8),
                          jnp.zeros((16,), jnp.int32)])

    t2m_loc, tp_loc = _sc_loc_kernel(
        model_parameters_t2m_mu, model_parameters_tp_cube_root_mu, md,
        w_t2m_loc, b_t2m_loc, w_tp_loc, b_tp_loc)

    t2m_scale, tp_scale = _sc_scale_kernel(
        model_parameters_t2m_sigma, model_parameters_tp_cube_root_sigma, md,
        w_t2m_scale, b_t2m_scale, w_tp_scale, b_tp_scale)

    return (t2m_loc, t2m_scale, tp_loc, tp_scale)
